# Initial kernel scaffold; baseline (speedup 1.0000x reference)
import jax
import jax.numpy as jnp
from jax.experimental import pallas as pl


def kernel(num, encodings, W, b):
    raise NotImplementedError("write your pallas kernel here")


# speedup vs baseline: 6.7667x; 6.7667x over previous
import jax
import jax.numpy as jnp
from jax.experimental import pallas as pl
from jax.experimental.pallas import tpu as pltpu
from jax.experimental.pallas import tpu_sc as plsc

ROWS = 100000
HIDDEN = 256
EMBED = 128
BATCH = 4096
HIST = 20
N_TOK = BATCH * HIST

ROW_BLK = 2000
GATHER_WIN = 128


def _proj_body(enc_ref, w_ref, b_ref, out_ref):
    out_ref[...] = jnp.dot(
        enc_ref[...], w_ref[...],
        preferred_element_type=jnp.float32,
        precision=jax.lax.Precision.HIGHEST,
    ) + b_ref[...]


def _project_table(enc, W, b):
    return pl.pallas_call(
        _proj_body,
        grid=(ROWS // ROW_BLK,),
        in_specs=[
            pl.BlockSpec((ROW_BLK, HIDDEN), lambda i: (i, 0)),
            pl.BlockSpec((HIDDEN, EMBED), lambda i: (0, 0)),
            pl.BlockSpec((1, EMBED), lambda i: (0, 0)),
        ],
        out_specs=pl.BlockSpec((ROW_BLK, EMBED), lambda i: (i, 0)),
        out_shape=jax.ShapeDtypeStruct((ROWS, EMBED), jnp.float32),
    )(enc, W, b.reshape(1, EMBED))


def _sc_gather(table, idx_flat):
    mesh = plsc.VectorSubcoreMesh(core_axis_name="core", subcore_axis_name="subcore")

    @pl.kernel(out_type=jax.ShapeDtypeStruct((N_TOK, EMBED), jnp.float32),
               mesh=mesh)
    def k(tab_hbm, i_hbm, o_hbm):
        def body(i_vmem, o_vmem):
            pltpu.sync_copy(tab_hbm.at[i_vmem.at[0]], o_vmem)

        pltpu.emit_pipeline(
            body,
            grid=(N_TOK // GATHER_WIN,),
            in_specs=[pl.BlockSpec((1, GATHER_WIN), index_map=lambda i: (0, i))],
            out_specs=[pl.BlockSpec((GATHER_WIN, EMBED), index_map=lambda i: (i, 0))],
            core_axis_name=("core", "subcore"),
            dimension_semantics=(pltpu.PARALLEL,),
        )(i_hbm, o_hbm)

    return k(table, idx_flat.reshape(1, N_TOK))


def kernel(num, encodings, W, b):
    P = _project_table(encodings, W, b)
    idx = num.astype(jnp.int32).reshape(-1)
    out = _sc_gather(P, idx)
    return out.reshape(BATCH, HIST, EMBED)


# speedup vs baseline: 7.3045x; 1.0795x over previous
import jax
import jax.numpy as jnp
from jax.experimental import pallas as pl
from jax.experimental.pallas import tpu as pltpu
from jax.experimental.pallas import tpu_sc as plsc

ROWS = 100000
HIDDEN = 256
EMBED = 128
BATCH = 4096
HIST = 20
N_TOK = BATCH * HIST

ROW_BLK = 2000
GATHER_WIN = 128


def _proj_body(enc_ref, w_ref, b_ref, out_ref):
    out_ref[...] = jnp.dot(
        enc_ref[...], w_ref[...],
        preferred_element_type=jnp.float32,
        precision=jax.lax.Precision.DEFAULT,
    ) + b_ref[...]


def _project_table(enc, W, b):
    return pl.pallas_call(
        _proj_body,
        grid=(ROWS // ROW_BLK,),
        in_specs=[
            pl.BlockSpec((ROW_BLK, HIDDEN), lambda i: (i, 0)),
            pl.BlockSpec((HIDDEN, EMBED), lambda i: (0, 0)),
            pl.BlockSpec((1, EMBED), lambda i: (0, 0)),
        ],
        out_specs=pl.BlockSpec((ROW_BLK, EMBED), lambda i: (i, 0)),
        out_shape=jax.ShapeDtypeStruct((ROWS, EMBED), jnp.float32),
    )(enc, W, b.reshape(1, EMBED))


BATCH_BLK = 8


def _sc_gather(table, idx):
    mesh = plsc.VectorSubcoreMesh(core_axis_name="core", subcore_axis_name="subcore")

    @pl.kernel(out_type=jax.ShapeDtypeStruct((BATCH, HIST, EMBED), jnp.float32),
               mesh=mesh)
    def k(tab_hbm, i_hbm, o_hbm):
        def body(i_vmem, o_vmem):
            for p in range(BATCH_BLK):
                pltpu.sync_copy(tab_hbm.at[i_vmem.at[p]], o_vmem.at[p])

        pltpu.emit_pipeline(
            body,
            grid=(BATCH // BATCH_BLK,),
            in_specs=[pl.BlockSpec((BATCH_BLK, HIST), index_map=lambda i: (i, 0))],
            out_specs=[pl.BlockSpec((BATCH_BLK, HIST, EMBED),
                                    index_map=lambda i: (i, 0, 0))],
            core_axis_name=("core", "subcore"),
            dimension_semantics=(pltpu.PARALLEL,),
        )(i_hbm, o_hbm)

    return k(table, idx)


def kernel(num, encodings, W, b):
    P = _project_table(encodings, W, b)
    idx = num.astype(jnp.int32)
    return _sc_gather(P, idx)


# speedup vs baseline: 10.6324x; 1.4556x over previous
import jax
import jax.numpy as jnp
from jax.experimental import pallas as pl
from jax.experimental.pallas import tpu as pltpu
from jax.experimental.pallas import tpu_sc as plsc

ROWS = 100000
HIDDEN = 256
EMBED = 128
BATCH = 4096
HIST = 20
N_TOK = BATCH * HIST

ROW_BLK = 2000
GATHER_WIN = 128


def _proj_body(enc_ref, w_ref, b_ref, out_ref):
    out_ref[...] = jnp.dot(
        enc_ref[...], w_ref[...],
        preferred_element_type=jnp.float32,
        precision=jax.lax.Precision.DEFAULT,
    ) + b_ref[...]


def _project_table(enc, W, b):
    return pl.pallas_call(
        _proj_body,
        grid=(ROWS // ROW_BLK,),
        in_specs=[
            pl.BlockSpec((ROW_BLK, HIDDEN), lambda i: (i, 0)),
            pl.BlockSpec((HIDDEN, EMBED), lambda i: (0, 0)),
            pl.BlockSpec((1, EMBED), lambda i: (0, 0)),
        ],
        out_specs=pl.BlockSpec((ROW_BLK, EMBED), lambda i: (i, 0)),
        out_shape=jax.ShapeDtypeStruct((ROWS, EMBED), jnp.float32),
    )(enc, W, b.reshape(1, EMBED))


BATCH_BLK = 16


def _sc_gather(table, idx):
    mesh = plsc.VectorSubcoreMesh(core_axis_name="core", subcore_axis_name="subcore")

    @pl.kernel(out_type=jax.ShapeDtypeStruct((BATCH, HIST, EMBED), jnp.float32),
               mesh=mesh,
               scratch_types=[pltpu.SemaphoreType.DMA])
    def k(tab_hbm, i_hbm, o_hbm, sem):
        def body(i_vmem, o_vmem):
            copies = [
                pltpu.async_copy(tab_hbm.at[i_vmem.at[p]], o_vmem.at[p], sem)
                for p in range(BATCH_BLK)
            ]
            for c in copies:
                c.wait()

        pltpu.emit_pipeline(
            body,
            grid=(BATCH // BATCH_BLK,),
            in_specs=[pl.BlockSpec((BATCH_BLK, HIST), index_map=lambda i: (i, 0))],
            out_specs=[pl.BlockSpec((BATCH_BLK, HIST, EMBED),
                                    index_map=lambda i: (i, 0, 0))],
            core_axis_name=("core", "subcore"),
            dimension_semantics=(pltpu.PARALLEL,),
        )(i_hbm, o_hbm)

    return k(table, idx)


def kernel(num, encodings, W, b):
    P = _project_table(encodings, W, b)
    idx = num.astype(jnp.int32)
    return _sc_gather(P, idx)


# speedup vs baseline: 11.8475x; 1.1143x over previous
import jax
import jax.numpy as jnp
from jax.experimental import pallas as pl
from jax.experimental.pallas import tpu as pltpu
from jax.experimental.pallas import tpu_sc as plsc

ROWS = 100000
HIDDEN = 256
EMBED = 128
BATCH = 4096
HIST = 20
N_TOK = BATCH * HIST

ROW_BLK = 5000
GATHER_WIN = 128


def _proj_body(enc_ref, w_ref, b_ref, out_ref):
    out_ref[...] = jnp.dot(
        enc_ref[...], w_ref[...],
        preferred_element_type=jnp.float32,
        precision=jax.lax.Precision.DEFAULT,
    ) + b_ref[...]


def _project_table(enc, W, b):
    return pl.pallas_call(
        _proj_body,
        grid=(ROWS // ROW_BLK,),
        in_specs=[
            pl.BlockSpec((ROW_BLK, HIDDEN), lambda i: (i, 0)),
            pl.BlockSpec((HIDDEN, EMBED), lambda i: (0, 0)),
            pl.BlockSpec((1, EMBED), lambda i: (0, 0)),
        ],
        out_specs=pl.BlockSpec((ROW_BLK, EMBED), lambda i: (i, 0)),
        out_shape=jax.ShapeDtypeStruct((ROWS, EMBED), jnp.float32),
    )(enc, W, b.reshape(1, EMBED))


BATCH_BLK = 16


def _sc_gather(table, idx):
    mesh = plsc.VectorSubcoreMesh(core_axis_name="core", subcore_axis_name="subcore")

    @pl.kernel(out_type=jax.ShapeDtypeStruct((BATCH, HIST, EMBED), jnp.float32),
               mesh=mesh,
               scratch_types=[pltpu.SemaphoreType.DMA])
    def k(tab_hbm, i_hbm, o_hbm, sem):
        def body(i_vmem, o_vmem):
            copies = [
                pltpu.async_copy(tab_hbm.at[i_vmem.at[p]], o_vmem.at[p], sem)
                for p in range(BATCH_BLK)
            ]
            for c in copies:
                c.wait()

        pltpu.emit_pipeline(
            body,
            grid=(BATCH // BATCH_BLK,),
            in_specs=[pl.BlockSpec((BATCH_BLK, HIST), index_map=lambda i: (i, 0))],
            out_specs=[pl.BlockSpec((BATCH_BLK, HIST, EMBED),
                                    index_map=lambda i: (i, 0, 0))],
            core_axis_name=("core", "subcore"),
            dimension_semantics=(pltpu.PARALLEL,),
        )(i_hbm, o_hbm)

    return k(table, idx)


def kernel(num, encodings, W, b):
    P = _project_table(encodings, W, b)
    idx = num.astype(jnp.int32)
    return _sc_gather(P, idx)


# speedup vs baseline: 12.0654x; 1.0184x over previous
import jax
import jax.numpy as jnp
from jax.experimental import pallas as pl
from jax.experimental.pallas import tpu as pltpu
from jax.experimental.pallas import tpu_sc as plsc

ROWS = 100000
HIDDEN = 256
EMBED = 128
BATCH = 4096
HIST = 20
N_TOK = BATCH * HIST

ROW_BLK = 10000
GATHER_WIN = 128


def _proj_body(enc_ref, w_ref, b_ref, out_ref):
    out_ref[...] = jnp.dot(
        enc_ref[...], w_ref[...],
        preferred_element_type=jnp.float32,
        precision=jax.lax.Precision.DEFAULT,
    ) + b_ref[...]


def _project_table(enc, W, b):
    return pl.pallas_call(
        _proj_body,
        grid=(ROWS // ROW_BLK,),
        in_specs=[
            pl.BlockSpec((ROW_BLK, HIDDEN), lambda i: (i, 0)),
            pl.BlockSpec((HIDDEN, EMBED), lambda i: (0, 0)),
            pl.BlockSpec((1, EMBED), lambda i: (0, 0)),
        ],
        out_specs=pl.BlockSpec((ROW_BLK, EMBED), lambda i: (i, 0)),
        out_shape=jax.ShapeDtypeStruct((ROWS, EMBED), jnp.float32),
    )(enc, W, b.reshape(1, EMBED))


BATCH_BLK = 16


def _sc_gather(table, idx):
    mesh = plsc.VectorSubcoreMesh(core_axis_name="core", subcore_axis_name="subcore")

    @pl.kernel(out_type=jax.ShapeDtypeStruct((BATCH, HIST, EMBED), jnp.float32),
               mesh=mesh,
               scratch_types=[pltpu.SemaphoreType.DMA])
    def k(tab_hbm, i_hbm, o_hbm, sem):
        def body(i_vmem, o_vmem):
            copies = [
                pltpu.async_copy(tab_hbm.at[i_vmem.at[p]], o_vmem.at[p], sem)
                for p in range(BATCH_BLK)
            ]
            for c in copies:
                c.wait()

        pltpu.emit_pipeline(
            body,
            grid=(BATCH // BATCH_BLK,),
            in_specs=[pl.BlockSpec((BATCH_BLK, HIST), index_map=lambda i: (i, 0))],
            out_specs=[pl.BlockSpec((BATCH_BLK, HIST, EMBED),
                                    index_map=lambda i: (i, 0, 0))],
            core_axis_name=("core", "subcore"),
            dimension_semantics=(pltpu.PARALLEL,),
        )(i_hbm, o_hbm)

    return k(table, idx)


def kernel(num, encodings, W, b):
    P = _project_table(encodings, W, b)
    idx = num.astype(jnp.int32)
    return _sc_gather(P, idx)
